# Initial kernel scaffold; baseline (speedup 1.0000x reference)
import jax
import jax.numpy as jnp
from jax.experimental import pallas as pl


def kernel(x, edge_index, batch, macro_edges, W1, b1, W2, b2, Wfc1, bfc1, Wfc2, bfc2, Wg1, bg1, Wg2, bg2, Wg3, bg3, gamma1, beta1, gamma2, beta2):
    raise NotImplementedError("write your pallas kernel here")


# speedup vs baseline: 7.2447x; 7.2447x over previous
import functools

import jax
import jax.numpy as jnp
from jax import lax
from jax.experimental import pallas as pl
from jax.experimental.pallas import tpu as pltpu
from jax.experimental.pallas import tpu_sc as plsc

N = 10000
E = 320000
G = 1000
ME = 16000
NP = 10240
EP = 327680
MEP = 16384
GP = 1024
NC = 2
NS = 16
CH = 128

_mesh = plsc.VectorSubcoreMesh(core_axis_name="c", subcore_axis_name="s",
                               num_cores=NC, num_subcores=NS)


def _zero_rows(buf, nrows, ncols):
    z = jnp.zeros((16,), jnp.float32)

    def body(i, _):
        for j in range(ncols // 16):
            buf[i, pl.ds(j * 16, 16)] = z
        return 0

    lax.fori_loop(0, nrows, body, 0)


def _fill1d(buf, n, val, dtype):
    v = jnp.full((16,), val, dtype)

    def body(i, _):
        buf[pl.ds(i * 16, 16)] = v
        return 0

    lax.fori_loop(0, n // 16, body, 0)


@functools.partial(
    pl.kernel,
    out_type=(
        jax.ShapeDtypeStruct((NC * NP,), jnp.float32),
        jax.ShapeDtypeStruct((NC * GP,), jnp.float32),
        jax.ShapeDtypeStruct((NC * GP,), jnp.float32),
        jax.ShapeDtypeStruct((NC * GP * GP,), jnp.float32),
    ),
    mesh=_mesh,
    scratch_types=[
        pltpu.VMEM((CH,), jnp.int32),
        pltpu.VMEM((CH,), jnp.int32),
        pltpu.VMEM((CH,), jnp.int32),
        pltpu.VMEM((64,), jnp.int32),
        pltpu.VMEM((CH,), jnp.float32),
        pltpu.VMEM((4096,), jnp.float32),
        pltpu.VMEM_SHARED((NP,), jnp.float32),
        pltpu.VMEM_SHARED((GP,), jnp.float32),
        pltpu.VMEM_SHARED((GP,), jnp.float32),
        pltpu.VMEM_SHARED((GP * GP,), jnp.float32),
    ],
)
def _hist_kernel(dste, srcm, dstm, batch, deg_out, degm_out, cnt_out, am_out,
                 dst_v, src_v, flat_v, b_v, ones_v, zeros_v,
                 acc_deg, acc_degm, acc_cnt, acc_am):
    c = lax.axis_index("c")
    s = lax.axis_index("s")
    wid = c * NS + s

    _fill1d(ones_v, CH, 1.0, jnp.float32)
    _fill1d(zeros_v, 4096, 0.0, jnp.float32)

    pltpu.sync_copy(zeros_v.at[pl.ds(0, NP // NS)], acc_deg.at[pl.ds(s * (NP // NS), NP // NS)])
    pltpu.sync_copy(zeros_v.at[pl.ds(0, GP // NS)], acc_degm.at[pl.ds(s * (GP // NS), GP // NS)])
    pltpu.sync_copy(zeros_v.at[pl.ds(0, GP // NS)], acc_cnt.at[pl.ds(s * (GP // NS), GP // NS)])

    def zam(j, _):
        pltpu.sync_copy(zeros_v, acc_am.at[pl.ds(s * (GP * GP // NS) + j * 4096, 4096)])
        return 0

    lax.fori_loop(0, GP * GP // NS // 4096, zam, 0)
    plsc.subcore_barrier()

    def micro(k, _):
        base = wid * (EP // (NC * NS)) + k * CH
        pltpu.sync_copy(dste.at[pl.ds(base, CH)], dst_v)
        pltpu.sync_copy(ones_v, acc_deg.at[dst_v], add=True)
        return 0

    lax.fori_loop(0, EP // (NC * NS) // CH, micro, 0)

    def macro(k, _):
        base = wid * (MEP // (NC * NS)) + k * CH
        pltpu.sync_copy(dstm.at[pl.ds(base, CH)], dst_v)
        pltpu.sync_copy(srcm.at[pl.ds(base, CH)], src_v)
        pltpu.sync_copy(ones_v, acc_degm.at[dst_v], add=True)
        for j in range(CH // 16):
            d16 = dst_v[pl.ds(j * 16, 16)]
            s16 = src_v[pl.ds(j * 16, 16)]
            flat_v[pl.ds(j * 16, 16)] = d16 * GP + s16
        pltpu.sync_copy(ones_v, acc_am.at[flat_v], add=True)
        return 0

    lax.fori_loop(0, MEP // (NC * NS) // CH, macro, 0)

    def bat(k, _):
        base = wid * (NP // (NC * NS)) + k * 64
        pltpu.sync_copy(batch.at[pl.ds(base, 64)], b_v)
        pltpu.sync_copy(ones_v.at[pl.ds(0, 64)], acc_cnt.at[b_v], add=True)
        return 0

    lax.fori_loop(0, NP // (NC * NS) // 64, bat, 0)
    plsc.subcore_barrier()

    pltpu.sync_copy(acc_deg.at[pl.ds(s * (NP // NS), NP // NS)],
                    zeros_v.at[pl.ds(0, NP // NS)])
    pltpu.sync_copy(zeros_v.at[pl.ds(0, NP // NS)],
                    deg_out.at[pl.ds(c * NP + s * (NP // NS), NP // NS)])
    pltpu.sync_copy(acc_degm.at[pl.ds(s * (GP // NS), GP // NS)],
                    ones_v.at[pl.ds(0, GP // NS)])
    pltpu.sync_copy(ones_v.at[pl.ds(0, GP // NS)],
                    degm_out.at[pl.ds(c * GP + s * (GP // NS), GP // NS)])
    pltpu.sync_copy(acc_cnt.at[pl.ds(s * (GP // NS), GP // NS)],
                    ones_v.at[pl.ds(0, GP // NS)])
    pltpu.sync_copy(ones_v.at[pl.ds(0, GP // NS)],
                    cnt_out.at[pl.ds(c * GP + s * (GP // NS), GP // NS)])

    def wam(j, _):
        off = s * (GP * GP // NS) + j * 4096
        pltpu.sync_copy(acc_am.at[pl.ds(off, 4096)], zeros_v)
        pltpu.sync_copy(zeros_v, am_out.at[pl.ds(c * GP * GP + off, 4096)])
        return 0

    lax.fori_loop(0, GP * GP // NS // 4096, wam, 0)


def _make_rowscatter(feature_split):
    trows = (2 * NP) if feature_split else NP
    chunks = EP // NS // CH if feature_split else EP // (NC * NS) // CH

    @functools.partial(
        pl.kernel,
        out_type=jax.ShapeDtypeStruct((NC * NP, 128), jnp.float32),
        mesh=_mesh,
        scratch_types=[
            pltpu.VMEM((CH,), jnp.int32),
            pltpu.VMEM((CH,), jnp.int32),
            pltpu.VMEM((CH,), jnp.int32),
            pltpu.VMEM((CH, 128), jnp.float32),
            pltpu.VMEM((CH, 128), jnp.float32),
            pltpu.VMEM_SHARED((NP, 128), jnp.float32),
            pltpu.SemaphoreType.DMA,
        ],
    )
    def rowscatter(table, srce, dste, out, src_v, gidx_v, dst_v, rows_v,
                   zrows_v, acc, sem):
        c = lax.axis_index("c")
        s = lax.axis_index("s")

        _zero_rows(zrows_v, CH, 128)
        rows_per_tile = NP // NS

        def zacc(j, _):
            pltpu.sync_copy(zrows_v, acc.at[pl.ds(s * rows_per_tile + j * CH, CH)])
            return 0

        lax.fori_loop(0, rows_per_tile // CH, zacc, 0)
        plsc.subcore_barrier()

        if feature_split:
            ebase0 = s * (EP // NS)
            goff = c * NP
        else:
            ebase0 = (c * NS + s) * (EP // (NC * NS))
            goff = 0

        def step(k, _):
            base = ebase0 + k * CH
            pltpu.sync_copy(srce.at[pl.ds(base, CH)], src_v)
            pltpu.sync_copy(dste.at[pl.ds(base, CH)], dst_v)
            for j in range(CH // 16):
                gidx_v[pl.ds(j * 16, 16)] = src_v[pl.ds(j * 16, 16)] + goff
            pltpu.async_copy(table.at[gidx_v], rows_v, sem).wait()
            pltpu.sync_copy(rows_v, acc.at[dst_v], add=True)
            return 0

        lax.fori_loop(0, chunks, step, 0)
        plsc.subcore_barrier()

        def wout(j, _):
            off = s * rows_per_tile + j * CH
            pltpu.sync_copy(acc.at[pl.ds(off, CH)], rows_v)
            pltpu.sync_copy(rows_v, out.at[pl.ds(c * NP + off, CH)])
            return 0

        lax.fori_loop(0, rows_per_tile // CH, wout, 0)

    return rowscatter


_rowscatter_feat = _make_rowscatter(True)
_rowscatter_edge = _make_rowscatter(False)


@functools.partial(
    pl.kernel,
    out_type=jax.ShapeDtypeStruct((NC * GP, 128), jnp.float32),
    mesh=_mesh,
    scratch_types=[
        pltpu.VMEM((64,), jnp.int32),
        pltpu.VMEM((64, 128), jnp.float32),
        pltpu.VMEM_SHARED((GP, 128), jnp.float32),
    ],
)
def _pool_kernel(h2, batch, out, b_v, rows_v, acc):
    c = lax.axis_index("c")
    s = lax.axis_index("s")

    _zero_rows(rows_v, 64, 128)
    pltpu.sync_copy(rows_v, acc.at[pl.ds(s * (GP // NS), GP // NS)])
    plsc.subcore_barrier()

    rows_per_tile = NP // (NC * NS)

    def step(k, _):
        base = (c * NS + s) * rows_per_tile + k * 64
        pltpu.sync_copy(h2.at[pl.ds(base, 64)], rows_v)
        pltpu.sync_copy(batch.at[pl.ds(base, 64)], b_v)
        pltpu.sync_copy(rows_v, acc.at[b_v], add=True)
        return 0

    lax.fori_loop(0, rows_per_tile // 64, step, 0)
    plsc.subcore_barrier()

    pltpu.sync_copy(acc.at[pl.ds(s * (GP // NS), GP // NS)], rows_v)
    pltpu.sync_copy(rows_v, out.at[pl.ds(c * GP + s * (GP // NS), GP // NS)])


BR = 1024
NBLK = NP // BR


def _k1_body(x_ref, w_ref, d0_ref, d1_ref, o_ref):
    dinv = lax.rsqrt(d0_ref[...] + d1_ref[...] + 1.0)
    t = jnp.dot(x_ref[...], w_ref[0], preferred_element_type=jnp.float32)
    o_ref[0] = t * dinv


def _tc_t1(x_p, w1r, deg0, deg1):
    return pl.pallas_call(
        _k1_body,
        grid=(2, NBLK),
        in_specs=[
            pl.BlockSpec((BR, 128), lambda c, i: (i, 0)),
            pl.BlockSpec((1, 128, 128), lambda c, i: (c, 0, 0)),
            pl.BlockSpec((BR, 1), lambda c, i: (i, 0)),
            pl.BlockSpec((BR, 1), lambda c, i: (i, 0)),
        ],
        out_specs=pl.BlockSpec((1, BR, 128), lambda c, i: (c, i, 0)),
        out_shape=jax.ShapeDtypeStruct((2, NP, 128), jnp.float32),
    )(x_p, w1r, deg0, deg1)


def _k3_body(t1_ref, a1_ref, d0_ref, d1_ref, w2_ref, b1_ref, o_ref):
    dinv = lax.rsqrt(d0_ref[...] + d1_ref[...] + 1.0)
    h1a = jnp.maximum((a1_ref[0] + t1_ref[0]) * dinv + b1_ref[0], 0.0)
    h1b = jnp.maximum((a1_ref[1] + t1_ref[1]) * dinv + b1_ref[1], 0.0)
    t2 = (jnp.dot(h1a, w2_ref[0], preferred_element_type=jnp.float32)
          + jnp.dot(h1b, w2_ref[1], preferred_element_type=jnp.float32))
    o_ref[...] = t2 * dinv


def _tc_t2(t1, agg1, deg0, deg1, w2r, b1r):
    return pl.pallas_call(
        _k3_body,
        grid=(NBLK,),
        in_specs=[
            pl.BlockSpec((2, BR, 128), lambda i: (0, i, 0)),
            pl.BlockSpec((2, BR, 128), lambda i: (0, i, 0)),
            pl.BlockSpec((BR, 1), lambda i: (i, 0)),
            pl.BlockSpec((BR, 1), lambda i: (i, 0)),
            pl.BlockSpec((2, 128, 128), lambda i: (0, 0, 0)),
            pl.BlockSpec((2, 128), lambda i: (0, 0)),
        ],
        out_specs=pl.BlockSpec((BR, 128), lambda i: (i, 0)),
        out_shape=jax.ShapeDtypeStruct((NP, 128), jnp.float32),
    )(t1, agg1, deg0, deg1, w2r, b1r)


def _k5_body(t2_ref, a2_ref, d0_ref, d1_ref, b2_ref, o_ref):
    dinv = lax.rsqrt(d0_ref[...] + d1_ref[...] + 1.0)
    o_ref[...] = jnp.maximum(
        (a2_ref[0] + a2_ref[1] + t2_ref[...]) * dinv + b2_ref[...], 0.0)


def _tc_h2(t2, agg2, deg0, deg1, b2r):
    return pl.pallas_call(
        _k5_body,
        grid=(NBLK,),
        in_specs=[
            pl.BlockSpec((BR, 128), lambda i: (i, 0)),
            pl.BlockSpec((2, BR, 128), lambda i: (0, i, 0)),
            pl.BlockSpec((BR, 1), lambda i: (i, 0)),
            pl.BlockSpec((BR, 1), lambda i: (i, 0)),
            pl.BlockSpec((1, 128), lambda i: (0, 0)),
        ],
        out_specs=pl.BlockSpec((BR, 128), lambda i: (i, 0)),
        out_shape=jax.ShapeDtypeStruct((NP, 128), jnp.float32),
    )(t2, agg2, deg0, deg1, b2r)


def _k7_body(p_ref, c0_ref, c1_ref, dm0_ref, dm1_ref, dmc0_ref, dmc1_ref,
             am_ref, wfc1_ref, bfc1_ref, wfc2_ref, bfc2_ref,
             wg1_ref, bg1_ref, wg2_ref, bg2_ref, wg3_ref, bg3_ref,
             ga1_ref, be1_ref, ga2_ref, be2_ref,
             loc_ref, ic_ref, g2_ref, hc_ref):
    cnt = c0_ref[...] + c1_ref[...]
    cntm = jnp.maximum(cnt, 1.0)
    mean = (p_ref[0] + p_ref[1]) / cntm
    loc = (jnp.dot(mean, wfc1_ref[...], preferred_element_type=jnp.float32)
           + bfc1_ref[...] * (cnt / cntm))
    loc_ref[...] = loc
    ic_ref[...] = (jnp.dot(loc, wfc2_ref[...], preferred_element_type=jnp.float32)
                   + bfc2_ref[...])

    dinv_r = lax.rsqrt(dm0_ref[...] + dm1_ref[...] + 1.0)
    dinv_c = lax.rsqrt(dmc0_ref[...] + dmc1_ref[...] + 1.0)
    rid = lax.broadcasted_iota(jnp.int32, (GP, GP), 0)
    cid = lax.broadcasted_iota(jnp.int32, (GP, GP), 1)
    eye = jnp.where(rid == cid, 1.0, 0.0)
    am = (am_ref[0] + am_ref[1] + eye) * (dinv_r * dinv_c)

    bnc = lax.rsqrt(jnp.float32(1.0 + 1e-5))
    u1 = jnp.dot(loc, wg1_ref[...], preferred_element_type=jnp.float32)
    g1 = jnp.dot(am, u1, preferred_element_type=jnp.float32) + bg1_ref[...]
    g1 = jnp.maximum(g1 * (ga1_ref[...] * bnc) + be1_ref[...], 0.0)
    u2 = jnp.dot(g1, wg2_ref[...], preferred_element_type=jnp.float32)
    g2 = jnp.dot(am, u2, preferred_element_type=jnp.float32) + bg2_ref[...]
    g2 = jnp.maximum(g2 * (ga2_ref[...] * bnc) + be2_ref[...], 0.0)
    g2_ref[...] = g2
    u3 = jnp.dot(g2, wg3_ref[...], preferred_element_type=jnp.float32)
    hc_ref[...] = jnp.dot(am, u3, preferred_element_type=jnp.float32) + bg3_ref[...]


def _tc_heads(pool, cnt0, cnt1, dm0, dm1, dmc0, dmc1, amp, wfc1, bfc1, wfc2,
              bfc2, wg1, bg1, wg2, bg2, wg3, bg3, ga1, be1, ga2, be2):
    return pl.pallas_call(
        _k7_body,
        out_shape=(
            jax.ShapeDtypeStruct((GP, 128), jnp.float32),
            jax.ShapeDtypeStruct((GP, 16), jnp.float32),
            jax.ShapeDtypeStruct((GP, 128), jnp.float32),
            jax.ShapeDtypeStruct((GP, 16), jnp.float32),
        ),
    )(pool, cnt0, cnt1, dm0, dm1, dmc0, dmc1, amp, wfc1, bfc1, wfc2, bfc2,
      wg1, bg1, wg2, bg2, wg3, bg3, ga1, be1, ga2, be2)


def kernel(x, edge_index, batch, macro_edges, W1, b1, W2, b2, Wfc1, bfc1,
           Wfc2, bfc2, Wg1, bg1, Wg2, bg2, Wg3, bg3, gamma1, beta1,
           gamma2, beta2):
    srce = jnp.pad(edge_index[0], (0, EP - E))
    dste = jnp.pad(edge_index[1], (0, EP - E), constant_values=N)
    srcm = jnp.pad(macro_edges[0], (0, MEP - ME))
    dstm = jnp.pad(macro_edges[1], (0, MEP - ME), constant_values=G)
    batch_p = jnp.pad(batch, (0, NP - N), constant_values=G)
    x_p = jnp.pad(x, ((0, NP - N), (0, 0)))

    degp, degmp, cntp, amp = _hist_kernel(dste, srcm, dstm, batch_p)
    deg0 = degp[:NP].reshape(NP, 1)
    deg1 = degp[NP:].reshape(NP, 1)
    dm0 = degmp[:GP].reshape(GP, 1)
    dm1 = degmp[GP:].reshape(GP, 1)
    dmc0 = degmp[:GP].reshape(1, GP)
    dmc1 = degmp[GP:].reshape(1, GP)
    cnt0 = cntp[:GP].reshape(GP, 1)
    cnt1 = cntp[GP:].reshape(GP, 1)

    w1r = W1.reshape(128, 2, 128).transpose(1, 0, 2)
    t1 = _tc_t1(x_p, w1r, deg0, deg1)

    agg1 = _rowscatter_feat(t1.reshape(2 * NP, 128), srce, dste)
    t2 = _tc_t2(t1, agg1.reshape(2, NP, 128), deg0, deg1,
                W2.reshape(2, 128, 128), b1.reshape(2, 128))

    agg2 = _rowscatter_edge(t2, srce, dste)
    h2 = _tc_h2(t2, agg2.reshape(2, NP, 128), deg0, deg1, b2.reshape(1, 128))

    pool = _pool_kernel(h2, batch_p)

    loc, ic, g2, hc = _tc_heads(
        pool.reshape(2, GP, 128), cnt0, cnt1, dm0, dm1, dmc0, dmc1,
        amp.reshape(2, GP, GP), Wfc1, bfc1.reshape(1, 128), Wfc2,
        bfc2.reshape(1, 16), Wg1, bg1.reshape(1, 128), Wg2,
        bg2.reshape(1, 128), Wg3, bg3.reshape(1, 16),
        gamma1.reshape(1, 128), beta1.reshape(1, 128),
        gamma2.reshape(1, 128), beta2.reshape(1, 128))

    return (hc[:G], ic[:G], loc[:G], g2[:G], jnp.float32(0.0))


# speedup vs baseline: 8.5137x; 1.1752x over previous
import functools

import jax
import jax.numpy as jnp
from jax import lax
from jax.experimental import pallas as pl
from jax.experimental.pallas import tpu as pltpu
from jax.experimental.pallas import tpu_sc as plsc

N = 10000
E = 320000
G = 1000
ME = 16000
NP = 10240
EP = 327680
MEP = 16384
GP = 1024
NC = 2
NS = 16
CH = 128

_mesh = plsc.VectorSubcoreMesh(core_axis_name="c", subcore_axis_name="s",
                               num_cores=NC, num_subcores=NS)


def _zero_rows(buf, nrows, ncols):
    z = jnp.zeros((16,), jnp.float32)

    def body(i, _):
        for j in range(ncols // 16):
            buf[i, pl.ds(j * 16, 16)] = z
        return 0

    lax.fori_loop(0, nrows, body, 0)


def _fill1d(buf, n, val, dtype):
    v = jnp.full((16,), val, dtype)

    def body(i, _):
        buf[pl.ds(i * 16, 16)] = v
        return 0

    lax.fori_loop(0, n // 16, body, 0)


@functools.partial(
    pl.kernel,
    out_type=(
        jax.ShapeDtypeStruct((NC * NP,), jnp.float32),
        jax.ShapeDtypeStruct((NC * GP,), jnp.float32),
        jax.ShapeDtypeStruct((NC * GP,), jnp.float32),
        jax.ShapeDtypeStruct((NC * GP * GP,), jnp.float32),
    ),
    mesh=_mesh,
    scratch_types=[
        pltpu.VMEM((CH,), jnp.int32),
        pltpu.VMEM((CH,), jnp.int32),
        pltpu.VMEM((CH,), jnp.int32),
        pltpu.VMEM((64,), jnp.int32),
        pltpu.VMEM((CH,), jnp.float32),
        pltpu.VMEM((4096,), jnp.float32),
        pltpu.VMEM_SHARED((NP,), jnp.float32),
        pltpu.VMEM_SHARED((GP,), jnp.float32),
        pltpu.VMEM_SHARED((GP,), jnp.float32),
        pltpu.VMEM_SHARED((GP * GP,), jnp.float32),
    ],
)
def _hist_kernel(dste, srcm, dstm, batch, deg_out, degm_out, cnt_out, am_out,
                 dst_v, src_v, flat_v, b_v, ones_v, zeros_v,
                 acc_deg, acc_degm, acc_cnt, acc_am):
    c = lax.axis_index("c")
    s = lax.axis_index("s")
    wid = c * NS + s

    _fill1d(ones_v, CH, 1.0, jnp.float32)
    _fill1d(zeros_v, 4096, 0.0, jnp.float32)

    pltpu.sync_copy(zeros_v.at[pl.ds(0, NP // NS)], acc_deg.at[pl.ds(s * (NP // NS), NP // NS)])
    pltpu.sync_copy(zeros_v.at[pl.ds(0, GP // NS)], acc_degm.at[pl.ds(s * (GP // NS), GP // NS)])
    pltpu.sync_copy(zeros_v.at[pl.ds(0, GP // NS)], acc_cnt.at[pl.ds(s * (GP // NS), GP // NS)])

    def zam(j, _):
        pltpu.sync_copy(zeros_v, acc_am.at[pl.ds(s * (GP * GP // NS) + j * 4096, 4096)])
        return 0

    lax.fori_loop(0, GP * GP // NS // 4096, zam, 0)
    plsc.subcore_barrier()

    def micro(k, _):
        base = wid * (EP // (NC * NS)) + k * CH
        pltpu.sync_copy(dste.at[pl.ds(base, CH)], dst_v)
        pltpu.sync_copy(ones_v, acc_deg.at[dst_v], add=True)
        return 0

    lax.fori_loop(0, EP // (NC * NS) // CH, micro, 0)

    def macro(k, _):
        base = wid * (MEP // (NC * NS)) + k * CH
        pltpu.sync_copy(dstm.at[pl.ds(base, CH)], dst_v)
        pltpu.sync_copy(srcm.at[pl.ds(base, CH)], src_v)
        pltpu.sync_copy(ones_v, acc_degm.at[dst_v], add=True)
        for j in range(CH // 16):
            d16 = dst_v[pl.ds(j * 16, 16)]
            s16 = src_v[pl.ds(j * 16, 16)]
            flat_v[pl.ds(j * 16, 16)] = d16 * GP + s16
        pltpu.sync_copy(ones_v, acc_am.at[flat_v], add=True)
        return 0

    lax.fori_loop(0, MEP // (NC * NS) // CH, macro, 0)

    def bat(k, _):
        base = wid * (NP // (NC * NS)) + k * 64
        pltpu.sync_copy(batch.at[pl.ds(base, 64)], b_v)
        pltpu.sync_copy(ones_v.at[pl.ds(0, 64)], acc_cnt.at[b_v], add=True)
        return 0

    lax.fori_loop(0, NP // (NC * NS) // 64, bat, 0)
    plsc.subcore_barrier()

    pltpu.sync_copy(acc_deg.at[pl.ds(s * (NP // NS), NP // NS)],
                    zeros_v.at[pl.ds(0, NP // NS)])
    pltpu.sync_copy(zeros_v.at[pl.ds(0, NP // NS)],
                    deg_out.at[pl.ds(c * NP + s * (NP // NS), NP // NS)])
    pltpu.sync_copy(acc_degm.at[pl.ds(s * (GP // NS), GP // NS)],
                    ones_v.at[pl.ds(0, GP // NS)])
    pltpu.sync_copy(ones_v.at[pl.ds(0, GP // NS)],
                    degm_out.at[pl.ds(c * GP + s * (GP // NS), GP // NS)])
    pltpu.sync_copy(acc_cnt.at[pl.ds(s * (GP // NS), GP // NS)],
                    ones_v.at[pl.ds(0, GP // NS)])
    pltpu.sync_copy(ones_v.at[pl.ds(0, GP // NS)],
                    cnt_out.at[pl.ds(c * GP + s * (GP // NS), GP // NS)])

    def wam(j, _):
        off = s * (GP * GP // NS) + j * 4096
        pltpu.sync_copy(acc_am.at[pl.ds(off, 4096)], zeros_v)
        pltpu.sync_copy(zeros_v, am_out.at[pl.ds(c * GP * GP + off, 4096)])
        return 0

    lax.fori_loop(0, GP * GP // NS // 4096, wam, 0)


NBUF = 2
WIN = 40


def _make_rowscatter(feature_split):
    chunks = EP // NS // CH if feature_split else EP // (NC * NS) // CH
    gpw = WIN // NBUF

    @functools.partial(
        pl.kernel,
        out_type=jax.ShapeDtypeStruct((NC * NP, 128), jnp.float32),
        mesh=_mesh,
        scratch_types=[
            pltpu.VMEM((WIN, CH), jnp.int32),
            pltpu.VMEM((WIN, CH), jnp.int32),
            [pltpu.VMEM((CH, 128), jnp.float32) for _ in range(NBUF)],
            [pltpu.SemaphoreType.DMA for _ in range(NBUF)],
            [pltpu.SemaphoreType.DMA for _ in range(NBUF)],
            pltpu.VMEM_SHARED((NP, 128), jnp.float32),
        ],
    )
    def rowscatter(table, srce2, dste2, out, gidx, didx, rows, gsems, ssems,
                   acc):
        c = lax.axis_index("c")
        s = lax.axis_index("s")
        rows_per_tile = NP // NS

        if feature_split:
            crow0 = s * (EP // NS // CH)
            goff = c * NP
        else:
            crow0 = (c * NS + s) * (EP // (NC * NS) // CH)
            goff = None

        _zero_rows(rows[0], CH, 128)

        def zacc(j, _):
            pltpu.sync_copy(rows[0], acc.at[pl.ds(s * rows_per_tile + j * CH, CH)])
            return 0

        lax.fori_loop(0, rows_per_tile // CH, zacc, 0)
        plsc.subcore_barrier()

        def gath(b, w):
            return pltpu.make_async_copy(table.at[gidx.at[w]], rows[b], gsems[b])

        def scat(b, w):
            return pltpu.make_async_copy(rows[b], acc.at[didx.at[w]], ssems[b])

        def group(g, _):
            at_win = (g % gpw) == 0

            @pl.when((g > 0) & at_win)
            def _():
                for b in range(NBUF):
                    scat(b, 0).wait()

            @pl.when(at_win)
            def _():
                wbase = pl.multiple_of(crow0 + g * NBUF, 8)
                pltpu.sync_copy(srce2.at[pl.ds(wbase, WIN)], gidx)
                pltpu.sync_copy(dste2.at[pl.ds(wbase, WIN)], didx)
                if goff is not None:
                    def addoff(i, _):
                        for j in range(CH // 16):
                            sl = pl.ds(j * 16, 16)
                            gidx[i, sl] = gidx[i, sl] + goff
                        return 0

                    lax.fori_loop(0, WIN, addoff, 0)

            for b in range(NBUF):
                @pl.when((g > 0) & jnp.logical_not(at_win))
                def _():
                    scat(b, 0).wait()

                gath(b, (g % gpw) * NBUF + b).start()
            for b in range(NBUF):
                w = (g % gpw) * NBUF + b
                gath(b, w).wait()
                scat(b, w).start(add=True)
            return 0

        lax.fori_loop(0, chunks // NBUF, group, 0)
        for b in range(NBUF):
            scat(b, 0).wait()
        plsc.subcore_barrier()

        def wout(j, _):
            off = s * rows_per_tile + j * CH
            pltpu.sync_copy(acc.at[pl.ds(off, CH)], rows[0])
            pltpu.sync_copy(rows[0], out.at[pl.ds(c * NP + off, CH)])
            return 0

        lax.fori_loop(0, rows_per_tile // CH, wout, 0)

    return rowscatter


_rowscatter_feat = _make_rowscatter(True)
_rowscatter_edge = _make_rowscatter(False)


@functools.partial(
    pl.kernel,
    out_type=jax.ShapeDtypeStruct((NC * GP, 128), jnp.float32),
    mesh=_mesh,
    scratch_types=[
        pltpu.VMEM((64,), jnp.int32),
        pltpu.VMEM((64, 128), jnp.float32),
        pltpu.VMEM_SHARED((GP, 128), jnp.float32),
    ],
)
def _pool_kernel(h2, batch, out, b_v, rows_v, acc):
    c = lax.axis_index("c")
    s = lax.axis_index("s")

    _zero_rows(rows_v, 64, 128)
    pltpu.sync_copy(rows_v, acc.at[pl.ds(s * (GP // NS), GP // NS)])
    plsc.subcore_barrier()

    rows_per_tile = NP // (NC * NS)

    def step(k, _):
        base = (c * NS + s) * rows_per_tile + k * 64
        pltpu.sync_copy(h2.at[pl.ds(base, 64)], rows_v)
        pltpu.sync_copy(batch.at[pl.ds(base, 64)], b_v)
        pltpu.sync_copy(rows_v, acc.at[b_v], add=True)
        return 0

    lax.fori_loop(0, rows_per_tile // 64, step, 0)
    plsc.subcore_barrier()

    pltpu.sync_copy(acc.at[pl.ds(s * (GP // NS), GP // NS)], rows_v)
    pltpu.sync_copy(rows_v, out.at[pl.ds(c * GP + s * (GP // NS), GP // NS)])


BR = 1024
NBLK = NP // BR


def _k1_body(x_ref, w_ref, d0_ref, d1_ref, o_ref):
    dinv = lax.rsqrt(d0_ref[...] + d1_ref[...] + 1.0)
    t = jnp.dot(x_ref[...], w_ref[0], preferred_element_type=jnp.float32)
    o_ref[0] = t * dinv


def _tc_t1(x_p, w1r, deg0, deg1):
    return pl.pallas_call(
        _k1_body,
        grid=(2, NBLK),
        in_specs=[
            pl.BlockSpec((BR, 128), lambda c, i: (i, 0)),
            pl.BlockSpec((1, 128, 128), lambda c, i: (c, 0, 0)),
            pl.BlockSpec((BR, 1), lambda c, i: (i, 0)),
            pl.BlockSpec((BR, 1), lambda c, i: (i, 0)),
        ],
        out_specs=pl.BlockSpec((1, BR, 128), lambda c, i: (c, i, 0)),
        out_shape=jax.ShapeDtypeStruct((2, NP, 128), jnp.float32),
    )(x_p, w1r, deg0, deg1)


def _k3_body(t1_ref, a1_ref, d0_ref, d1_ref, w2_ref, b1_ref, o_ref):
    dinv = lax.rsqrt(d0_ref[...] + d1_ref[...] + 1.0)
    h1a = jnp.maximum((a1_ref[0] + t1_ref[0]) * dinv + b1_ref[0], 0.0)
    h1b = jnp.maximum((a1_ref[1] + t1_ref[1]) * dinv + b1_ref[1], 0.0)
    t2 = (jnp.dot(h1a, w2_ref[0], preferred_element_type=jnp.float32)
          + jnp.dot(h1b, w2_ref[1], preferred_element_type=jnp.float32))
    o_ref[...] = t2 * dinv


def _tc_t2(t1, agg1, deg0, deg1, w2r, b1r):
    return pl.pallas_call(
        _k3_body,
        grid=(NBLK,),
        in_specs=[
            pl.BlockSpec((2, BR, 128), lambda i: (0, i, 0)),
            pl.BlockSpec((2, BR, 128), lambda i: (0, i, 0)),
            pl.BlockSpec((BR, 1), lambda i: (i, 0)),
            pl.BlockSpec((BR, 1), lambda i: (i, 0)),
            pl.BlockSpec((2, 128, 128), lambda i: (0, 0, 0)),
            pl.BlockSpec((2, 128), lambda i: (0, 0)),
        ],
        out_specs=pl.BlockSpec((BR, 128), lambda i: (i, 0)),
        out_shape=jax.ShapeDtypeStruct((NP, 128), jnp.float32),
    )(t1, agg1, deg0, deg1, w2r, b1r)


def _k5_body(t2_ref, a2_ref, d0_ref, d1_ref, b2_ref, o_ref):
    dinv = lax.rsqrt(d0_ref[...] + d1_ref[...] + 1.0)
    o_ref[...] = jnp.maximum(
        (a2_ref[0] + a2_ref[1] + t2_ref[...]) * dinv + b2_ref[...], 0.0)


def _tc_h2(t2, agg2, deg0, deg1, b2r):
    return pl.pallas_call(
        _k5_body,
        grid=(NBLK,),
        in_specs=[
            pl.BlockSpec((BR, 128), lambda i: (i, 0)),
            pl.BlockSpec((2, BR, 128), lambda i: (0, i, 0)),
            pl.BlockSpec((BR, 1), lambda i: (i, 0)),
            pl.BlockSpec((BR, 1), lambda i: (i, 0)),
            pl.BlockSpec((1, 128), lambda i: (0, 0)),
        ],
        out_specs=pl.BlockSpec((BR, 128), lambda i: (i, 0)),
        out_shape=jax.ShapeDtypeStruct((NP, 128), jnp.float32),
    )(t2, agg2, deg0, deg1, b2r)


def _k7_body(p_ref, c0_ref, c1_ref, dm0_ref, dm1_ref, dmc0_ref, dmc1_ref,
             am_ref, wfc1_ref, bfc1_ref, wfc2_ref, bfc2_ref,
             wg1_ref, bg1_ref, wg2_ref, bg2_ref, wg3_ref, bg3_ref,
             ga1_ref, be1_ref, ga2_ref, be2_ref,
             loc_ref, ic_ref, g2_ref, hc_ref):
    cnt = c0_ref[...] + c1_ref[...]
    cntm = jnp.maximum(cnt, 1.0)
    mean = (p_ref[0] + p_ref[1]) / cntm
    loc = (jnp.dot(mean, wfc1_ref[...], preferred_element_type=jnp.float32)
           + bfc1_ref[...] * (cnt / cntm))
    loc_ref[...] = loc
    ic_ref[...] = (jnp.dot(loc, wfc2_ref[...], preferred_element_type=jnp.float32)
                   + bfc2_ref[...])

    dinv_r = lax.rsqrt(dm0_ref[...] + dm1_ref[...] + 1.0)
    dinv_c = lax.rsqrt(dmc0_ref[...] + dmc1_ref[...] + 1.0)
    rid = lax.broadcasted_iota(jnp.int32, (GP, GP), 0)
    cid = lax.broadcasted_iota(jnp.int32, (GP, GP), 1)
    eye = jnp.where(rid == cid, 1.0, 0.0)
    am = (am_ref[0] + am_ref[1] + eye) * (dinv_r * dinv_c)

    bnc = lax.rsqrt(jnp.float32(1.0 + 1e-5))
    u1 = jnp.dot(loc, wg1_ref[...], preferred_element_type=jnp.float32)
    g1 = jnp.dot(am, u1, preferred_element_type=jnp.float32) + bg1_ref[...]
    g1 = jnp.maximum(g1 * (ga1_ref[...] * bnc) + be1_ref[...], 0.0)
    u2 = jnp.dot(g1, wg2_ref[...], preferred_element_type=jnp.float32)
    g2 = jnp.dot(am, u2, preferred_element_type=jnp.float32) + bg2_ref[...]
    g2 = jnp.maximum(g2 * (ga2_ref[...] * bnc) + be2_ref[...], 0.0)
    g2_ref[...] = g2
    u3 = jnp.dot(g2, wg3_ref[...], preferred_element_type=jnp.float32)
    hc_ref[...] = jnp.dot(am, u3, preferred_element_type=jnp.float32) + bg3_ref[...]


def _tc_heads(pool, cnt0, cnt1, dm0, dm1, dmc0, dmc1, amp, wfc1, bfc1, wfc2,
              bfc2, wg1, bg1, wg2, bg2, wg3, bg3, ga1, be1, ga2, be2):
    return pl.pallas_call(
        _k7_body,
        out_shape=(
            jax.ShapeDtypeStruct((GP, 128), jnp.float32),
            jax.ShapeDtypeStruct((GP, 16), jnp.float32),
            jax.ShapeDtypeStruct((GP, 128), jnp.float32),
            jax.ShapeDtypeStruct((GP, 16), jnp.float32),
        ),
    )(pool, cnt0, cnt1, dm0, dm1, dmc0, dmc1, amp, wfc1, bfc1, wfc2, bfc2,
      wg1, bg1, wg2, bg2, wg3, bg3, ga1, be1, ga2, be2)


def kernel(x, edge_index, batch, macro_edges, W1, b1, W2, b2, Wfc1, bfc1,
           Wfc2, bfc2, Wg1, bg1, Wg2, bg2, Wg3, bg3, gamma1, beta1,
           gamma2, beta2):
    srce = jnp.pad(edge_index[0], (0, EP - E))
    dste = jnp.pad(edge_index[1], (0, EP - E), constant_values=N)
    srcm = jnp.pad(macro_edges[0], (0, MEP - ME))
    dstm = jnp.pad(macro_edges[1], (0, MEP - ME), constant_values=G)
    batch_p = jnp.pad(batch, (0, NP - N), constant_values=G)
    x_p = jnp.pad(x, ((0, NP - N), (0, 0)))

    degp, degmp, cntp, amp = _hist_kernel(dste, srcm, dstm, batch_p)
    deg0 = degp[:NP].reshape(NP, 1)
    deg1 = degp[NP:].reshape(NP, 1)
    dm0 = degmp[:GP].reshape(GP, 1)
    dm1 = degmp[GP:].reshape(GP, 1)
    dmc0 = degmp[:GP].reshape(1, GP)
    dmc1 = degmp[GP:].reshape(1, GP)
    cnt0 = cntp[:GP].reshape(GP, 1)
    cnt1 = cntp[GP:].reshape(GP, 1)

    w1r = W1.reshape(128, 2, 128).transpose(1, 0, 2)
    t1 = _tc_t1(x_p, w1r, deg0, deg1)

    srce2 = srce.reshape(EP // CH, CH)
    dste2 = dste.reshape(EP // CH, CH)
    agg1 = _rowscatter_feat(t1.reshape(2 * NP, 128), srce2, dste2)
    t2 = _tc_t2(t1, agg1.reshape(2, NP, 128), deg0, deg1,
                W2.reshape(2, 128, 128), b1.reshape(2, 128))

    agg2 = _rowscatter_edge(t2, srce2, dste2)
    h2 = _tc_h2(t2, agg2.reshape(2, NP, 128), deg0, deg1, b2.reshape(1, 128))

    pool = _pool_kernel(h2, batch_p)

    loc, ic, g2, hc = _tc_heads(
        pool.reshape(2, GP, 128), cnt0, cnt1, dm0, dm1, dmc0, dmc1,
        amp.reshape(2, GP, GP), Wfc1, bfc1.reshape(1, 128), Wfc2,
        bfc2.reshape(1, 16), Wg1, bg1.reshape(1, 128), Wg2,
        bg2.reshape(1, 128), Wg3, bg3.reshape(1, 16),
        gamma1.reshape(1, 128), beta1.reshape(1, 128),
        gamma2.reshape(1, 128), beta2.reshape(1, 128))

    return (hc[:G], ic[:G], loc[:G], g2[:G], jnp.float32(0.0))


# speedup vs baseline: 9.4834x; 1.1139x over previous
import functools

import jax
import jax.numpy as jnp
from jax import lax
from jax.experimental import pallas as pl
from jax.experimental.pallas import tpu as pltpu
from jax.experimental.pallas import tpu_sc as plsc

N = 10000
E = 320000
G = 1000
ME = 16000
NP = 10240
EP = 327680
MEP = 16384
GP = 1024
NC = 2
NS = 16
CH = 128

_mesh = plsc.VectorSubcoreMesh(core_axis_name="c", subcore_axis_name="s",
                               num_cores=NC, num_subcores=NS)


def _zero_rows(buf, nrows, ncols):
    z = jnp.zeros((16,), jnp.float32)

    def body(i, _):
        for j in range(ncols // 16):
            buf[i, pl.ds(j * 16, 16)] = z
        return 0

    lax.fori_loop(0, nrows, body, 0)


def _fill1d(buf, n, val, dtype):
    v = jnp.full((16,), val, dtype)

    def body(i, _):
        buf[pl.ds(i * 16, 16)] = v
        return 0

    lax.fori_loop(0, n // 16, body, 0)


MCH = 64
MROWS = MEP // 64 // (NC * NS)
BHP = 16384
BROWS = BHP // 64 // (NC * NS)
ECH_T = EP // (NC * NS) // CH


@functools.partial(
    pl.kernel,
    out_type=(
        jax.ShapeDtypeStruct((NC * NP,), jnp.float32),
        jax.ShapeDtypeStruct((NC * GP,), jnp.float32),
        jax.ShapeDtypeStruct((NC * GP,), jnp.float32),
        jax.ShapeDtypeStruct((NC * GP * GP,), jnp.float32),
    ),
    mesh=_mesh,
    scratch_types=[
        pltpu.VMEM((ECH_T, CH), jnp.int32),
        pltpu.VMEM((MROWS, MCH), jnp.int32),
        pltpu.VMEM((MROWS, MCH), jnp.int32),
        pltpu.VMEM((MROWS, MCH), jnp.int32),
        pltpu.VMEM((BROWS, MCH), jnp.int32),
        pltpu.VMEM((CH,), jnp.float32),
        [pltpu.VMEM((8192,), jnp.float32) for _ in range(2)],
        [pltpu.SemaphoreType.DMA for _ in range(4)],
        [pltpu.SemaphoreType.DMA for _ in range(2)],
        pltpu.VMEM_SHARED((NP,), jnp.float32),
        pltpu.VMEM_SHARED((GP,), jnp.float32),
        pltpu.VMEM_SHARED((GP,), jnp.float32),
        pltpu.VMEM_SHARED((GP * GP,), jnp.float32),
    ],
)
def _hist_kernel(dste2, srcm2, dstm2, batch2, deg_out, degm_out, cnt_out,
                 am_out, dst2d, msrc, mdst, mflat, b2d, ones_v, wb, sems,
                 wsems, acc_deg, acc_degm, acc_cnt, acc_am):
    c = lax.axis_index("c")
    s = lax.axis_index("s")
    wid = c * NS + s

    _fill1d(ones_v, CH, 1.0, jnp.float32)
    _fill1d(wb[0], 8192, 0.0, jnp.float32)

    pltpu.sync_copy(dste2.at[pl.ds(wid * ECH_T, ECH_T)], dst2d)
    pltpu.sync_copy(srcm2.at[pl.ds(wid * MROWS, MROWS)], msrc)
    pltpu.sync_copy(dstm2.at[pl.ds(wid * MROWS, MROWS)], mdst)
    pltpu.sync_copy(batch2.at[pl.ds(wid * BROWS, BROWS)], b2d)
    for i in range(MROWS):
        for j in range(MCH // 16):
            sl = pl.ds(j * 16, 16)
            mflat[i, sl] = mdst[i, sl] * GP + msrc[i, sl]

    zn = GP * GP // NS // 8192
    for j in range(zn):
        pltpu.make_async_copy(
            wb[0], acc_am.at[pl.ds(s * (GP * GP // NS) + j * 8192, 8192)],
            sems[0]).start()
    pltpu.make_async_copy(
        wb[0].at[pl.ds(0, NP // NS)],
        acc_deg.at[pl.ds(s * (NP // NS), NP // NS)], sems[1]).start()
    pltpu.make_async_copy(
        wb[0].at[pl.ds(0, GP // NS)],
        acc_degm.at[pl.ds(s * (GP // NS), GP // NS)], sems[2]).start()
    pltpu.make_async_copy(
        wb[0].at[pl.ds(0, GP // NS)],
        acc_cnt.at[pl.ds(s * (GP // NS), GP // NS)], sems[3]).start()
    for j in range(zn):
        pltpu.make_async_copy(
            wb[0], acc_am.at[pl.ds(s * (GP * GP // NS) + j * 8192, 8192)],
            sems[0]).wait()
    pltpu.make_async_copy(
        wb[0].at[pl.ds(0, NP // NS)],
        acc_deg.at[pl.ds(s * (NP // NS), NP // NS)], sems[1]).wait()
    pltpu.make_async_copy(
        wb[0].at[pl.ds(0, GP // NS)],
        acc_degm.at[pl.ds(s * (GP // NS), GP // NS)], sems[2]).wait()
    pltpu.make_async_copy(
        wb[0].at[pl.ds(0, GP // NS)],
        acc_cnt.at[pl.ds(s * (GP // NS), GP // NS)], sems[3]).wait()
    plsc.subcore_barrier()

    def dsca(b, k):
        return pltpu.make_async_copy(ones_v, acc_deg.at[dst2d.at[k]], sems[b])

    def micro(g, _):
        for b in range(4):
            @pl.when(g > 0)
            def _():
                dsca(b, 0).wait()

            dsca(b, g * 4 + b).start(add=True)
        return 0

    lax.fori_loop(0, ECH_T // 4, micro, 0)
    for b in range(4):
        dsca(b, 0).wait()

    o64 = ones_v.at[pl.ds(0, MCH)]
    descs = ([pltpu.make_async_copy(o64, acc_degm.at[mdst.at[i]], sems[i % 2])
              for i in range(MROWS)]
             + [pltpu.make_async_copy(o64, acc_am.at[mflat.at[i]], sems[2 + i % 2])
                for i in range(MROWS)]
             + [pltpu.make_async_copy(o64, acc_cnt.at[b2d.at[i]], sems[i % 2])
                for i in range(BROWS)])
    for d in descs:
        d.start(add=True)
    for d in descs:
        d.wait()
    plsc.subcore_barrier()

    pltpu.sync_copy(acc_deg.at[pl.ds(s * (NP // NS), NP // NS)],
                    wb[1].at[pl.ds(0, NP // NS)])
    pltpu.sync_copy(wb[1].at[pl.ds(0, NP // NS)],
                    deg_out.at[pl.ds(c * NP + s * (NP // NS), NP // NS)])
    pltpu.sync_copy(acc_degm.at[pl.ds(s * (GP // NS), GP // NS)],
                    wb[1].at[pl.ds(0, GP // NS)])
    pltpu.sync_copy(wb[1].at[pl.ds(0, GP // NS)],
                    degm_out.at[pl.ds(c * GP + s * (GP // NS), GP // NS)])
    pltpu.sync_copy(acc_cnt.at[pl.ds(s * (GP // NS), GP // NS)],
                    wb[1].at[pl.ds(0, GP // NS)])
    pltpu.sync_copy(wb[1].at[pl.ds(0, GP // NS)],
                    cnt_out.at[pl.ds(c * GP + s * (GP // NS), GP // NS)])

    def wdesc(b, j):
        off = s * (GP * GP // NS) + j * 8192
        return pltpu.make_async_copy(
            wb[b], am_out.at[pl.ds(c * GP * GP + off, 8192)], wsems[b])

    for j in range(GP * GP // NS // 8192):
        b = j % 2
        if j > 1:
            wdesc(b, j - 2).wait()
        pltpu.sync_copy(acc_am.at[pl.ds(s * (GP * GP // NS) + j * 8192, 8192)],
                        wb[b])
        wdesc(b, j).start()
    for j in (GP * GP // NS // 8192 - 2, GP * GP // NS // 8192 - 1):
        wdesc(j % 2, j).wait()


NBUF = 2
WIN = 40


def _make_rowscatter(feature_split):
    chunks = EP // NS // CH if feature_split else EP // (NC * NS) // CH
    gpw = WIN // NBUF

    @functools.partial(
        pl.kernel,
        out_type=jax.ShapeDtypeStruct((NC * NP, 128), jnp.float32),
        mesh=_mesh,
        scratch_types=[
            pltpu.VMEM((WIN, CH), jnp.int32),
            pltpu.VMEM((WIN, CH), jnp.int32),
            [pltpu.VMEM((CH, 128), jnp.float32) for _ in range(NBUF)],
            [pltpu.SemaphoreType.DMA for _ in range(NBUF)],
            [pltpu.SemaphoreType.DMA for _ in range(NBUF)],
            pltpu.VMEM_SHARED((NP, 128), jnp.float32),
        ],
    )
    def rowscatter(table, srce2, dste2, out, gidx, didx, rows, gsems, ssems,
                   acc):
        c = lax.axis_index("c")
        s = lax.axis_index("s")
        rows_per_tile = NP // NS

        if feature_split:
            crow0 = s * (EP // NS // CH)
            goff = c * NP
        else:
            crow0 = (c * NS + s) * (EP // (NC * NS) // CH)
            goff = None

        _zero_rows(rows[0], CH, 128)

        def zdesc(j):
            return pltpu.make_async_copy(
                rows[0], acc.at[pl.ds(s * rows_per_tile + j * CH, CH)],
                gsems[0])

        for j in range(rows_per_tile // CH):
            zdesc(j).start()
        for j in range(rows_per_tile // CH):
            zdesc(j).wait()
        plsc.subcore_barrier()

        def gath(b, w):
            return pltpu.make_async_copy(table.at[gidx.at[w]], rows[b], gsems[b])

        def scat(b, w):
            return pltpu.make_async_copy(rows[b], acc.at[didx.at[w]], ssems[b])

        def group(g, _):
            at_win = (g % gpw) == 0

            @pl.when((g > 0) & at_win)
            def _():
                for b in range(NBUF):
                    scat(b, 0).wait()

            @pl.when(at_win)
            def _():
                wbase = pl.multiple_of(crow0 + g * NBUF, 8)
                pltpu.sync_copy(srce2.at[pl.ds(wbase, WIN)], gidx)
                pltpu.sync_copy(dste2.at[pl.ds(wbase, WIN)], didx)
                if goff is not None:
                    def addoff(i, _):
                        for j in range(CH // 16):
                            sl = pl.ds(j * 16, 16)
                            gidx[i, sl] = gidx[i, sl] + goff
                        return 0

                    lax.fori_loop(0, WIN, addoff, 0)

            for b in range(NBUF):
                @pl.when((g > 0) & jnp.logical_not(at_win))
                def _():
                    scat(b, 0).wait()

                gath(b, (g % gpw) * NBUF + b).start()
            for b in range(NBUF):
                w = (g % gpw) * NBUF + b
                gath(b, w).wait()
                scat(b, w).start(add=True)
            return 0

        lax.fori_loop(0, chunks // NBUF, group, 0)
        for b in range(NBUF):
            scat(b, 0).wait()
        plsc.subcore_barrier()

        def wdesc(b, j):
            off = s * rows_per_tile + j * CH
            return pltpu.make_async_copy(
                rows[b], out.at[pl.ds(c * NP + off, CH)], gsems[b])

        nw = rows_per_tile // CH
        for j in range(nw):
            b = j % NBUF
            if j >= NBUF:
                wdesc(b, j - NBUF).wait()
            pltpu.sync_copy(acc.at[pl.ds(s * rows_per_tile + j * CH, CH)],
                            rows[b])
            wdesc(b, j).start()
        for j in range(nw - NBUF, nw):
            wdesc(j % NBUF, j).wait()

    return rowscatter


_rowscatter_feat = _make_rowscatter(True)
_rowscatter_edge = _make_rowscatter(False)


@functools.partial(
    pl.kernel,
    out_type=jax.ShapeDtypeStruct((NC * GP, 128), jnp.float32),
    mesh=_mesh,
    scratch_types=[
        pltpu.VMEM((64,), jnp.int32),
        pltpu.VMEM((64, 128), jnp.float32),
        pltpu.VMEM_SHARED((GP, 128), jnp.float32),
    ],
)
def _pool_kernel(h2, batch, out, b_v, rows_v, acc):
    c = lax.axis_index("c")
    s = lax.axis_index("s")

    _zero_rows(rows_v, 64, 128)
    pltpu.sync_copy(rows_v, acc.at[pl.ds(s * (GP // NS), GP // NS)])
    plsc.subcore_barrier()

    rows_per_tile = NP // (NC * NS)

    def step(k, _):
        base = (c * NS + s) * rows_per_tile + k * 64
        pltpu.sync_copy(h2.at[pl.ds(base, 64)], rows_v)
        pltpu.sync_copy(batch.at[pl.ds(base, 64)], b_v)
        pltpu.sync_copy(rows_v, acc.at[b_v], add=True)
        return 0

    lax.fori_loop(0, rows_per_tile // 64, step, 0)
    plsc.subcore_barrier()

    pltpu.sync_copy(acc.at[pl.ds(s * (GP // NS), GP // NS)], rows_v)
    pltpu.sync_copy(rows_v, out.at[pl.ds(c * GP + s * (GP // NS), GP // NS)])


BR = 1024
NBLK = NP // BR


def _k1_body(x_ref, w_ref, d0_ref, d1_ref, o_ref):
    dinv = lax.rsqrt(d0_ref[...] + d1_ref[...] + 1.0)
    t = jnp.dot(x_ref[...], w_ref[0], preferred_element_type=jnp.float32)
    o_ref[0] = t * dinv


def _tc_t1(x_p, w1r, deg0, deg1):
    return pl.pallas_call(
        _k1_body,
        grid=(2, NBLK),
        in_specs=[
            pl.BlockSpec((BR, 128), lambda c, i: (i, 0)),
            pl.BlockSpec((1, 128, 128), lambda c, i: (c, 0, 0)),
            pl.BlockSpec((BR, 1), lambda c, i: (i, 0)),
            pl.BlockSpec((BR, 1), lambda c, i: (i, 0)),
        ],
        out_specs=pl.BlockSpec((1, BR, 128), lambda c, i: (c, i, 0)),
        out_shape=jax.ShapeDtypeStruct((2, NP, 128), jnp.float32),
    )(x_p, w1r, deg0, deg1)


def _k3_body(t1_ref, a1_ref, d0_ref, d1_ref, w2_ref, b1_ref, o_ref):
    dinv = lax.rsqrt(d0_ref[...] + d1_ref[...] + 1.0)
    h1a = jnp.maximum((a1_ref[0] + t1_ref[0]) * dinv + b1_ref[0], 0.0)
    h1b = jnp.maximum((a1_ref[1] + t1_ref[1]) * dinv + b1_ref[1], 0.0)
    t2 = (jnp.dot(h1a, w2_ref[0], preferred_element_type=jnp.float32)
          + jnp.dot(h1b, w2_ref[1], preferred_element_type=jnp.float32))
    o_ref[...] = t2 * dinv


def _tc_t2(t1, agg1, deg0, deg1, w2r, b1r):
    return pl.pallas_call(
        _k3_body,
        grid=(NBLK,),
        in_specs=[
            pl.BlockSpec((2, BR, 128), lambda i: (0, i, 0)),
            pl.BlockSpec((2, BR, 128), lambda i: (0, i, 0)),
            pl.BlockSpec((BR, 1), lambda i: (i, 0)),
            pl.BlockSpec((BR, 1), lambda i: (i, 0)),
            pl.BlockSpec((2, 128, 128), lambda i: (0, 0, 0)),
            pl.BlockSpec((2, 128), lambda i: (0, 0)),
        ],
        out_specs=pl.BlockSpec((BR, 128), lambda i: (i, 0)),
        out_shape=jax.ShapeDtypeStruct((NP, 128), jnp.float32),
    )(t1, agg1, deg0, deg1, w2r, b1r)


def _k5_body(t2_ref, a2_ref, d0_ref, d1_ref, b2_ref, o_ref):
    dinv = lax.rsqrt(d0_ref[...] + d1_ref[...] + 1.0)
    o_ref[...] = jnp.maximum(
        (a2_ref[0] + a2_ref[1] + t2_ref[...]) * dinv + b2_ref[...], 0.0)


def _tc_h2(t2, agg2, deg0, deg1, b2r):
    return pl.pallas_call(
        _k5_body,
        grid=(NBLK,),
        in_specs=[
            pl.BlockSpec((BR, 128), lambda i: (i, 0)),
            pl.BlockSpec((2, BR, 128), lambda i: (0, i, 0)),
            pl.BlockSpec((BR, 1), lambda i: (i, 0)),
            pl.BlockSpec((BR, 1), lambda i: (i, 0)),
            pl.BlockSpec((1, 128), lambda i: (0, 0)),
        ],
        out_specs=pl.BlockSpec((BR, 128), lambda i: (i, 0)),
        out_shape=jax.ShapeDtypeStruct((NP, 128), jnp.float32),
    )(t2, agg2, deg0, deg1, b2r)


def _k7_body(p_ref, c0_ref, c1_ref, dm0_ref, dm1_ref, dmc0_ref, dmc1_ref,
             am_ref, wfc1_ref, bfc1_ref, wfc2_ref, bfc2_ref,
             wg1_ref, bg1_ref, wg2_ref, bg2_ref, wg3_ref, bg3_ref,
             ga1_ref, be1_ref, ga2_ref, be2_ref,
             loc_ref, ic_ref, g2_ref, hc_ref):
    cnt = c0_ref[...] + c1_ref[...]
    cntm = jnp.maximum(cnt, 1.0)
    mean = (p_ref[0] + p_ref[1]) / cntm
    loc = (jnp.dot(mean, wfc1_ref[...], preferred_element_type=jnp.float32)
           + bfc1_ref[...] * (cnt / cntm))
    loc_ref[...] = loc
    ic_ref[...] = (jnp.dot(loc, wfc2_ref[...], preferred_element_type=jnp.float32)
                   + bfc2_ref[...])

    dinv_r = lax.rsqrt(dm0_ref[...] + dm1_ref[...] + 1.0)
    dinv_c = lax.rsqrt(dmc0_ref[...] + dmc1_ref[...] + 1.0)
    rid = lax.broadcasted_iota(jnp.int32, (GP, GP), 0)
    cid = lax.broadcasted_iota(jnp.int32, (GP, GP), 1)
    eye = jnp.where(rid == cid, 1.0, 0.0)
    am = (am_ref[0] + am_ref[1] + eye) * (dinv_r * dinv_c)

    bnc = lax.rsqrt(jnp.float32(1.0 + 1e-5))
    u1 = jnp.dot(loc, wg1_ref[...], preferred_element_type=jnp.float32)
    g1 = jnp.dot(am, u1, preferred_element_type=jnp.float32) + bg1_ref[...]
    g1 = jnp.maximum(g1 * (ga1_ref[...] * bnc) + be1_ref[...], 0.0)
    u2 = jnp.dot(g1, wg2_ref[...], preferred_element_type=jnp.float32)
    g2 = jnp.dot(am, u2, preferred_element_type=jnp.float32) + bg2_ref[...]
    g2 = jnp.maximum(g2 * (ga2_ref[...] * bnc) + be2_ref[...], 0.0)
    g2_ref[...] = g2
    u3 = jnp.dot(g2, wg3_ref[...], preferred_element_type=jnp.float32)
    hc_ref[...] = jnp.dot(am, u3, preferred_element_type=jnp.float32) + bg3_ref[...]


def _tc_heads(pool, cnt0, cnt1, dm0, dm1, dmc0, dmc1, amp, wfc1, bfc1, wfc2,
              bfc2, wg1, bg1, wg2, bg2, wg3, bg3, ga1, be1, ga2, be2):
    return pl.pallas_call(
        _k7_body,
        out_shape=(
            jax.ShapeDtypeStruct((GP, 128), jnp.float32),
            jax.ShapeDtypeStruct((GP, 16), jnp.float32),
            jax.ShapeDtypeStruct((GP, 128), jnp.float32),
            jax.ShapeDtypeStruct((GP, 16), jnp.float32),
        ),
    )(pool, cnt0, cnt1, dm0, dm1, dmc0, dmc1, amp, wfc1, bfc1, wfc2, bfc2,
      wg1, bg1, wg2, bg2, wg3, bg3, ga1, be1, ga2, be2)


def kernel(x, edge_index, batch, macro_edges, W1, b1, W2, b2, Wfc1, bfc1,
           Wfc2, bfc2, Wg1, bg1, Wg2, bg2, Wg3, bg3, gamma1, beta1,
           gamma2, beta2):
    srce = jnp.pad(edge_index[0], (0, EP - E))
    dste = jnp.pad(edge_index[1], (0, EP - E), constant_values=N)
    srcm2 = jnp.pad(macro_edges[0], (0, MEP - ME)).reshape(MEP // 64, 64)
    dstm2 = jnp.pad(macro_edges[1], (0, MEP - ME),
                    constant_values=G).reshape(MEP // 64, 64)
    batch_p = jnp.pad(batch, (0, NP - N), constant_values=G)
    batch2h = jnp.pad(batch, (0, BHP - N),
                      constant_values=G).reshape(BHP // 64, 64)
    x_p = jnp.pad(x, ((0, NP - N), (0, 0)))

    dste2h = dste.reshape(EP // CH, CH)
    degp, degmp, cntp, amp = _hist_kernel(dste2h, srcm2, dstm2, batch2h)
    deg0 = degp[:NP].reshape(NP, 1)
    deg1 = degp[NP:].reshape(NP, 1)
    dm0 = degmp[:GP].reshape(GP, 1)
    dm1 = degmp[GP:].reshape(GP, 1)
    dmc0 = degmp[:GP].reshape(1, GP)
    dmc1 = degmp[GP:].reshape(1, GP)
    cnt0 = cntp[:GP].reshape(GP, 1)
    cnt1 = cntp[GP:].reshape(GP, 1)

    w1r = W1.reshape(128, 2, 128).transpose(1, 0, 2)
    t1 = _tc_t1(x_p, w1r, deg0, deg1)

    srce2 = srce.reshape(EP // CH, CH)
    dste2 = dste.reshape(EP // CH, CH)
    agg1 = _rowscatter_feat(t1.reshape(2 * NP, 128), srce2, dste2)
    t2 = _tc_t2(t1, agg1.reshape(2, NP, 128), deg0, deg1,
                W2.reshape(2, 128, 128), b1.reshape(2, 128))

    agg2 = _rowscatter_edge(t2, srce2, dste2)
    h2 = _tc_h2(t2, agg2.reshape(2, NP, 128), deg0, deg1, b2.reshape(1, 128))

    pool = _pool_kernel(h2, batch_p)

    loc, ic, g2, hc = _tc_heads(
        pool.reshape(2, GP, 128), cnt0, cnt1, dm0, dm1, dmc0, dmc1,
        amp.reshape(2, GP, GP), Wfc1, bfc1.reshape(1, 128), Wfc2,
        bfc2.reshape(1, 16), Wg1, bg1.reshape(1, 128), Wg2,
        bg2.reshape(1, 128), Wg3, bg3.reshape(1, 16),
        gamma1.reshape(1, 128), beta1.reshape(1, 128),
        gamma2.reshape(1, 128), beta2.reshape(1, 128))

    return (hc[:G], ic[:G], loc[:G], g2[:G], jnp.float32(0.0))
